# baseline (speedup 1.0000x reference)
import functools

import jax
import jax.numpy as jnp
from jax import lax
from jax.experimental import pallas as pl
from jax.experimental.pallas import tpu as pltpu
from jax.experimental.pallas import tpu_sc as plsc

VOCAB = 1000000
D = 64
N_TOK = 10
B = 4096
S = 200
P0 = 8
NP = S - P0
NBAND = NP // 8

NC = 2
NS = 16
NW = NC * NS

_mesh = plsc.VectorSubcoreMesh(core_axis_name="c", subcore_axis_name="s")


@functools.partial(
    pl.kernel,
    mesh=_mesh,
    out_type=jax.ShapeDtypeStruct((S, D // 8, B // 128, 8, 128), jnp.float32),
    scratch_types=[
        pltpu.VMEM((8, 128), jnp.int32),
        pltpu.VMEM((8, 128, D), jnp.float32),
        pltpu.VMEM((2, 8, 1, 8, 128), jnp.float32),
        pltpu.VMEM((16, D), jnp.float32),
        pltpu.SemaphoreType.DMA((8,)),
        pltpu.SemaphoreType.DMA((2,)),
    ],
    compiler_params=pltpu.CompilerParams(
        use_tc_tiling_on_sc=False, needs_layout_passes=False),
)
def _soft_embed(tok_hbm, wte_hbm, soft_hbm, out_hbm,
                idx_v, pbuf, tbuf, soft_v, gsem, ssem):
    wid = lax.axis_index("s") * NC + lax.axis_index("c")

    pltpu.sync_copy(soft_hbm, soft_v)
    iota = lax.iota(jnp.int32, 16)
    ridx = [jnp.int32(16 * g) + iota for g in range(8)]

    def transpose_into(j, r):
        for c in range(D):
            cidx = jnp.broadcast_to(jnp.int32(c), (16,))
            for g in range(8):
                v = plsc.load_gather(pbuf.at[r], [ridx[g], cidx])
                tbuf[j, c // 8, 0, c % 8, pl.ds(16 * g, 16)] = v

    def band_body(t, carry):
        p0 = P0 + 8 * t
        pltpu.sync_copy(tok_hbm.at[pl.ds(p0, 8), pl.ds(wid * 128, 128)], idx_v)
        for r in range(8):
            pltpu.async_copy(wte_hbm.at[idx_v.at[r]], pbuf.at[r], gsem.at[r])

        def pos_body(r, carry2):
            j = lax.rem(r, 2)
            g_done = pltpu.make_async_copy(
                wte_hbm.at[idx_v.at[0]], pbuf.at[r], gsem.at[r])
            g_done.wait()

            @pl.when(t * 8 + r >= 2)
            def _():
                pltpu.make_async_copy(
                    tbuf.at[j], out_hbm.at[p0, :, pl.ds(wid, 1)],
                    ssem.at[j]).wait()
            transpose_into(j, r)
            pltpu.async_copy(
                tbuf.at[j], out_hbm.at[p0 + r, :, pl.ds(wid, 1)], ssem.at[j])
            return carry2

        lax.fori_loop(0, 8, pos_body, 0)
        return carry

    lax.fori_loop(0, NBAND, band_body, 0)

    for j in range(2):
        pltpu.make_async_copy(
            tbuf.at[j], out_hbm.at[0, :, pl.ds(wid, 1)], ssem.at[j]).wait()

    def soft_store(p, carry):
        pidx = jnp.broadcast_to(p.astype(jnp.int32), (16,))
        for d in range(D):
            didx = jnp.broadcast_to(jnp.int32(d), (16,))
            v = plsc.load_gather(soft_v, [pidx, didx])
            for g in range(8):
                tbuf[0, d // 8, 0, d % 8, pl.ds(16 * g, 16)] = v
        pltpu.sync_copy(tbuf.at[0], out_hbm.at[p, :, pl.ds(wid, 1)])
        return carry
    lax.fori_loop(0, N_TOK, soft_store, 0)


def kernel(tokens, wte_weight, soft_embedding_weight):
    tok_t = jnp.transpose(tokens.astype(jnp.int32))
    softp = jnp.pad(soft_embedding_weight, ((0, 16 - N_TOK), (0, 0)))
    res = _soft_embed(tok_t, wte_weight, softp)
    return jnp.transpose(res, (2, 4, 0, 1, 3)).reshape(B, S, D)


# speedup vs baseline: 1.4329x; 1.4329x over previous
import functools

import jax
import jax.numpy as jnp
from jax import lax
from jax.experimental import pallas as pl
from jax.experimental.pallas import tpu as pltpu
from jax.experimental.pallas import tpu_sc as plsc

VOCAB = 1000000
D = 64
N_TOK = 10
B = 4096
S = 200
P0 = 8
NP = S - P0
NBAND = NP // 8

NC = 2
NS = 16
NW = NC * NS

_mesh = plsc.VectorSubcoreMesh(core_axis_name="c", subcore_axis_name="s")


@functools.partial(
    pl.kernel,
    mesh=_mesh,
    out_type=jax.ShapeDtypeStruct((S, D // 8, B // 128, 8, 128), jnp.float32),
    scratch_types=[
        pltpu.VMEM((8, 128), jnp.int32),
        pltpu.VMEM((8, 128, D), jnp.float32),
        pltpu.VMEM((2, 8, 1, 8, 128), jnp.float32),
        pltpu.VMEM((16, D), jnp.float32),
        pltpu.SemaphoreType.DMA((8,)),
        pltpu.SemaphoreType.DMA((2,)),
    ],
    compiler_params=pltpu.CompilerParams(
        use_tc_tiling_on_sc=False, needs_layout_passes=False),
)
def _soft_embed(tok_hbm, wte_hbm, soft_hbm, out_hbm,
                idx_v, pbuf, tbuf, soft_v, gsem, ssem):
    wid = lax.axis_index("s") * NC + lax.axis_index("c")

    pltpu.sync_copy(soft_hbm, soft_v)
    iota = lax.iota(jnp.int32, 16)
    ridx = [jnp.int32(16 * g) + iota for g in range(8)]

    zeros16 = jnp.broadcast_to(jnp.int32(0), (16,))

    def transpose_into(j, r):
        for c in range(D):
            dsel = lax.bitwise_and(jnp.int32(c) + iota, jnp.int32(63))
            dt = lax.shift_right_logical(dsel, 3)
            di = lax.bitwise_and(dsel, jnp.int32(7))
            for g in range(8):
                v = plsc.load_gather(pbuf.at[r], [ridx[g], dsel])
                plsc.store_scatter(tbuf.at[j], [dt, zeros16, di, ridx[g]], v)

    def band_body(t, carry):
        p0 = P0 + 8 * t
        pltpu.sync_copy(tok_hbm.at[pl.ds(p0, 8), pl.ds(wid * 128, 128)], idx_v)
        for r in range(8):
            pltpu.async_copy(wte_hbm.at[idx_v.at[r]], pbuf.at[r], gsem.at[r])

        def pos_body(r, carry2):
            j = lax.rem(r, 2)
            g_done = pltpu.make_async_copy(
                wte_hbm.at[idx_v.at[0]], pbuf.at[r], gsem.at[r])
            g_done.wait()

            @pl.when(t * 8 + r >= 2)
            def _():
                pltpu.make_async_copy(
                    tbuf.at[j], out_hbm.at[p0, :, pl.ds(wid, 1)],
                    ssem.at[j]).wait()
            transpose_into(j, r)
            pltpu.async_copy(
                tbuf.at[j], out_hbm.at[p0 + r, :, pl.ds(wid, 1)], ssem.at[j])
            return carry2

        lax.fori_loop(0, 8, pos_body, 0)
        return carry

    lax.fori_loop(0, NBAND, band_body, 0)

    for j in range(2):
        pltpu.make_async_copy(
            tbuf.at[j], out_hbm.at[0, :, pl.ds(wid, 1)], ssem.at[j]).wait()

    def soft_store(p, carry):
        pidx = jnp.broadcast_to(p.astype(jnp.int32), (16,))
        for d in range(D):
            didx = jnp.broadcast_to(jnp.int32(d), (16,))
            v = plsc.load_gather(soft_v, [pidx, didx])
            for g in range(8):
                tbuf[0, d // 8, 0, d % 8, pl.ds(16 * g, 16)] = v
        pltpu.sync_copy(tbuf.at[0], out_hbm.at[p, :, pl.ds(wid, 1)])
        return carry
    lax.fori_loop(0, N_TOK, soft_store, 0)


def kernel(tokens, wte_weight, soft_embedding_weight):
    tok_t = jnp.transpose(tokens.astype(jnp.int32))
    softp = jnp.pad(soft_embedding_weight, ((0, 16 - N_TOK), (0, 0)))
    res = _soft_embed(tok_t, wte_weight, softp)
    return jnp.transpose(res, (2, 4, 0, 1, 3)).reshape(B, S, D)


# speedup vs baseline: 1.5785x; 1.1017x over previous
import functools

import jax
import jax.numpy as jnp
from jax import lax
from jax.experimental import pallas as pl
from jax.experimental.pallas import tpu as pltpu
from jax.experimental.pallas import tpu_sc as plsc

VOCAB = 1000000
D = 64
N_TOK = 10
B = 4096
S = 200
P0 = 8
NP = S - P0
NBAND = NP // 8

NC = 2
NS = 16
NW = NC * NS

_mesh = plsc.VectorSubcoreMesh(core_axis_name="c", subcore_axis_name="s")


@functools.partial(
    pl.kernel,
    mesh=_mesh,
    out_type=jax.ShapeDtypeStruct((S, D // 8, B // 128, 8, 128), jnp.float32),
    scratch_types=[
        pltpu.VMEM((2, 8, 128), jnp.int32),
        pltpu.VMEM((8, 128, D), jnp.float32),
        pltpu.VMEM((2, 8, 1, 8, 128), jnp.float32),
        pltpu.VMEM((16, D), jnp.float32),
        pltpu.SemaphoreType.DMA((8,)),
        pltpu.SemaphoreType.DMA((2,)),
    ],
    compiler_params=pltpu.CompilerParams(
        use_tc_tiling_on_sc=False, needs_layout_passes=False),
)
def _soft_embed(tok_hbm, wte_hbm, soft_hbm, out_hbm,
                idx_v, pbuf, tbuf, soft_v, gsem, ssem):
    wid = lax.axis_index("s") * NC + lax.axis_index("c")

    pltpu.sync_copy(soft_hbm, soft_v)
    iota = lax.iota(jnp.int32, 16)
    ridx = [jnp.int32(16 * g) + iota for g in range(8)]

    zeros16 = jnp.broadcast_to(jnp.int32(0), (16,))

    def transpose_into(j, r):
        for c in range(D):
            dsel = lax.bitwise_and(jnp.int32(c) + iota, jnp.int32(63))
            dt = lax.shift_right_logical(dsel, 3)
            di = lax.bitwise_and(dsel, jnp.int32(7))
            for g in range(8):
                v = plsc.load_gather(pbuf.at[r], [ridx[g], dsel])
                plsc.store_scatter(tbuf.at[j], [dt, zeros16, di, ridx[g]], v)

    pltpu.sync_copy(tok_hbm.at[pl.ds(P0, 8), pl.ds(wid * 128, 128)],
                    idx_v.at[0])
    for r in range(8):
        pltpu.async_copy(wte_hbm.at[idx_v.at[0, r]], pbuf.at[r], gsem.at[r])

    def band_body(t, carry):
        p0 = P0 + 8 * t
        par = lax.rem(t, 2)
        nxt = lax.rem(t + 1, 2)

        @pl.when(t + 1 < NBAND)
        def _():
            pltpu.sync_copy(
                tok_hbm.at[pl.ds(p0 + 8, 8), pl.ds(wid * 128, 128)],
                idx_v.at[nxt])

        def pos_body(r, carry2):
            j = lax.rem(r, 2)
            pltpu.make_async_copy(
                wte_hbm.at[idx_v.at[0, 0]], pbuf.at[r], gsem.at[r]).wait()

            @pl.when(t * 8 + r >= 2)
            def _():
                pltpu.make_async_copy(
                    tbuf.at[j], out_hbm.at[p0, :, pl.ds(wid, 1)],
                    ssem.at[j]).wait()
            transpose_into(j, r)
            pltpu.async_copy(
                tbuf.at[j], out_hbm.at[p0 + r, :, pl.ds(wid, 1)], ssem.at[j])

            @pl.when(t + 1 < NBAND)
            def _():
                pltpu.async_copy(
                    wte_hbm.at[idx_v.at[nxt, r]], pbuf.at[r], gsem.at[r])
            return carry2

        lax.fori_loop(0, 8, pos_body, 0)
        return carry

    lax.fori_loop(0, NBAND, band_body, 0)

    for j in range(2):
        pltpu.make_async_copy(
            tbuf.at[j], out_hbm.at[0, :, pl.ds(wid, 1)], ssem.at[j]).wait()

    def soft_store(p, carry):
        pidx = jnp.broadcast_to(p.astype(jnp.int32), (16,))
        for d in range(D):
            didx = jnp.broadcast_to(jnp.int32(d), (16,))
            v = plsc.load_gather(soft_v, [pidx, didx])
            for g in range(8):
                tbuf[0, d // 8, 0, d % 8, pl.ds(16 * g, 16)] = v
        pltpu.sync_copy(tbuf.at[0], out_hbm.at[p, :, pl.ds(wid, 1)])
        return carry
    lax.fori_loop(0, N_TOK, soft_store, 0)


def kernel(tokens, wte_weight, soft_embedding_weight):
    tok_t = jnp.transpose(tokens.astype(jnp.int32))
    softp = jnp.pad(soft_embedding_weight, ((0, 16 - N_TOK), (0, 0)))
    res = _soft_embed(tok_t, wte_weight, softp)
    return jnp.transpose(res, (2, 4, 0, 1, 3)).reshape(B, S, D)


# speedup vs baseline: 2.3811x; 1.5084x over previous
import functools

import jax
import jax.numpy as jnp
from jax import lax
from jax.experimental import pallas as pl
from jax.experimental.pallas import tpu as pltpu
from jax.experimental.pallas import tpu_sc as plsc

VOCAB = 1000000
D = 64
N_TOK = 10
B = 4096
S = 200
P0 = 8
NP = S - P0
NBAND = NP // 8

NC = 2
NS = 16
NW = NC * NS

_mesh = plsc.VectorSubcoreMesh(core_axis_name="c", subcore_axis_name="s")


@functools.partial(
    pl.kernel,
    mesh=_mesh,
    out_type=jax.ShapeDtypeStruct((S, D // 8, B // 128, 8, 128), jnp.float32),
    scratch_types=[
        pltpu.VMEM((2, 8, 128), jnp.int32),
        pltpu.VMEM((8, 128, D), jnp.float32),
        pltpu.VMEM((2, 8, 1, 8, 128), jnp.float32),
        pltpu.VMEM((16, D), jnp.float32),
        pltpu.SemaphoreType.DMA((8,)),
        pltpu.SemaphoreType.DMA((2,)),
    ],
    compiler_params=pltpu.CompilerParams(
        use_tc_tiling_on_sc=False, needs_layout_passes=False),
)
def _soft_embed(tok_hbm, wte_hbm, soft_hbm, out_hbm,
                idx_v, pbuf, tbuf, soft_v, gsem, ssem):
    wid = lax.axis_index("s") * NC + lax.axis_index("c")

    pltpu.sync_copy(soft_hbm, soft_v)
    iota = lax.iota(jnp.int32, 16)
    ridx = [jnp.int32(16 * g) + iota for g in range(8)]

    zeros16 = jnp.broadcast_to(jnp.int32(0), (16,))

    def transpose_into(j, r):
        for c in range(D):
            dsel = lax.bitwise_and(jnp.int32(c) + iota, jnp.int32(63))
            dt = lax.shift_right_logical(dsel, 3)
            di = lax.bitwise_and(dsel, jnp.int32(7))
            vs = [plsc.load_gather(pbuf.at[r], [ridx[g], dsel])
                  for g in range(8)]
            for g in range(8):
                plsc.store_scatter(tbuf.at[j], [dt, zeros16, di, ridx[g]],
                                   vs[g])

    pltpu.sync_copy(tok_hbm.at[pl.ds(P0, 8), pl.ds(wid * 128, 128)],
                    idx_v.at[0])
    for r in range(8):
        pltpu.async_copy(wte_hbm.at[idx_v.at[0, r]], pbuf.at[r], gsem.at[r])

    def band_body(t, carry):
        p0 = P0 + 8 * t
        par = lax.rem(t, 2)
        nxt = lax.rem(t + 1, 2)

        @pl.when(t + 1 < NBAND)
        def _():
            pltpu.sync_copy(
                tok_hbm.at[pl.ds(p0 + 8, 8), pl.ds(wid * 128, 128)],
                idx_v.at[nxt])

        def pos_body(r, carry2):
            j = lax.rem(r, 2)
            pltpu.make_async_copy(
                wte_hbm.at[idx_v.at[0, 0]], pbuf.at[r], gsem.at[r]).wait()

            @pl.when(t * 8 + r >= 2)
            def _():
                pltpu.make_async_copy(
                    tbuf.at[j], out_hbm.at[p0, :, pl.ds(wid, 1)],
                    ssem.at[j]).wait()
            transpose_into(j, r)
            pltpu.async_copy(
                tbuf.at[j], out_hbm.at[p0 + r, :, pl.ds(wid, 1)], ssem.at[j])

            @pl.when(t + 1 < NBAND)
            def _():
                pltpu.async_copy(
                    wte_hbm.at[idx_v.at[nxt, r]], pbuf.at[r], gsem.at[r])
            return carry2

        lax.fori_loop(0, 8, pos_body, 0)
        return carry

    lax.fori_loop(0, NBAND, band_body, 0)

    for j in range(2):
        pltpu.make_async_copy(
            tbuf.at[j], out_hbm.at[0, :, pl.ds(wid, 1)], ssem.at[j]).wait()

    def soft_store(p, carry):
        pidx = jnp.broadcast_to(p.astype(jnp.int32), (16,))
        for d in range(D):
            didx = jnp.broadcast_to(jnp.int32(d), (16,))
            v = plsc.load_gather(soft_v, [pidx, didx])
            for g in range(8):
                tbuf[0, d // 8, 0, d % 8, pl.ds(16 * g, 16)] = v
        pltpu.sync_copy(tbuf.at[0], out_hbm.at[p, :, pl.ds(wid, 1)])
        return carry
    lax.fori_loop(0, N_TOK, soft_store, 0)


def kernel(tokens, wte_weight, soft_embedding_weight):
    tok_t = jnp.transpose(tokens.astype(jnp.int32))
    softp = jnp.pad(soft_embedding_weight, ((0, 16 - N_TOK), (0, 0)))
    res = _soft_embed(tok_t, wte_weight, softp)
    return jnp.transpose(res, (2, 4, 0, 1, 3)).reshape(B, S, D)


# speedup vs baseline: 2.3830x; 1.0008x over previous
import functools

import jax
import jax.numpy as jnp
from jax import lax
from jax.experimental import pallas as pl
from jax.experimental.pallas import tpu as pltpu
from jax.experimental.pallas import tpu_sc as plsc

VOCAB = 1000000
D = 64
N_TOK = 10
B = 4096
S = 200
P0 = 8
NP = S - P0
NBAND = NP // 8

NC = 2
NS = 16
NW = NC * NS

_mesh = plsc.VectorSubcoreMesh(core_axis_name="c", subcore_axis_name="s")


@functools.partial(
    pl.kernel,
    mesh=_mesh,
    out_type=jax.ShapeDtypeStruct((S, D // 8, B // 128, 8, 128), jnp.float32),
    scratch_types=[
        pltpu.VMEM((2, 8, 128), jnp.int32),
        pltpu.VMEM((8, 128, D), jnp.float32),
        pltpu.VMEM((2, 8, 1, 8, 128), jnp.float32),
        pltpu.VMEM((16, D), jnp.float32),
        pltpu.SemaphoreType.DMA((8,)),
        pltpu.SemaphoreType.DMA((2,)),
    ],
    compiler_params=pltpu.CompilerParams(
        use_tc_tiling_on_sc=False, needs_layout_passes=False),
)
def _soft_embed(tok_hbm, wte_hbm, soft_hbm, out_hbm,
                idx_v, pbuf, tbuf, soft_v, gsem, ssem):
    wid = lax.axis_index("s") * NC + lax.axis_index("c")

    pltpu.sync_copy(soft_hbm, soft_v)
    iota = lax.iota(jnp.int32, 16)
    ridx = [jnp.int32(16 * g) + iota for g in range(8)]

    zeros16 = jnp.broadcast_to(jnp.int32(0), (16,))

    def transpose_into(j, r):
        for c in range(D):
            dsel = lax.bitwise_and(jnp.int32(c) + iota, jnp.int32(63))
            dt = lax.shift_right_logical(dsel, 3)
            di = lax.bitwise_and(dsel, jnp.int32(7))
            vs = [plsc.load_gather(pbuf.at[r], [ridx[g], dsel])
                  for g in range(8)]
            for g in range(8):
                plsc.store_scatter(tbuf.at[j], [dt, zeros16, di, ridx[g]],
                                   vs[g])

    pltpu.sync_copy(tok_hbm.at[pl.ds(P0, 8), pl.ds(wid * 128, 128)],
                    idx_v.at[0])
    for r in range(8):
        pltpu.async_copy(wte_hbm.at[idx_v.at[0, r]], pbuf.at[r], gsem.at[r])

    def band_body(t, carry):
        p0 = P0 + 8 * t
        nxt = lax.rem(t + 1, 2)

        @pl.when(t + 1 < NBAND)
        def _():
            pltpu.sync_copy(
                tok_hbm.at[pl.ds(p0 + 8, 8), pl.ds(wid * 128, 128)],
                idx_v.at[nxt])

        def pos_body(r, carry2):
            j = lax.rem(r, 2)
            pltpu.make_async_copy(
                wte_hbm.at[idx_v.at[0, 0]], pbuf.at[r], gsem.at[r]).wait()

            @pl.when(t * 8 + r >= 2)
            def _():
                pltpu.make_async_copy(
                    tbuf.at[j], out_hbm.at[p0, :, pl.ds(wid, 1)],
                    ssem.at[j]).wait()
            transpose_into(j, r)
            pltpu.async_copy(
                tbuf.at[j], out_hbm.at[p0 + r, :, pl.ds(wid, 1)], ssem.at[j])

            @pl.when(t + 1 < NBAND)
            def _():
                pltpu.async_copy(
                    wte_hbm.at[idx_v.at[nxt, r]], pbuf.at[r], gsem.at[r])
            return carry2

        lax.fori_loop(0, 8, pos_body, 0)
        return carry

    lax.fori_loop(0, NBAND, band_body, 0)

    for j in range(2):
        pltpu.make_async_copy(
            tbuf.at[j], out_hbm.at[0, :, pl.ds(wid, 1)], ssem.at[j]).wait()

    def soft_store(p, carry):
        pidx = jnp.broadcast_to(p.astype(jnp.int32), (16,))
        for d in range(D):
            didx = jnp.broadcast_to(jnp.int32(d), (16,))
            v = plsc.load_gather(soft_v, [pidx, didx])
            for g in range(8):
                tbuf[0, d // 8, 0, d % 8, pl.ds(16 * g, 16)] = v
        pltpu.sync_copy(tbuf.at[0], out_hbm.at[p, :, pl.ds(wid, 1)])
        return carry
    lax.fori_loop(0, N_TOK, soft_store, 0)


def kernel(tokens, wte_weight, soft_embedding_weight):
    tok_t = jnp.transpose(tokens.astype(jnp.int32))
    softp = jnp.pad(soft_embedding_weight, ((0, 16 - N_TOK), (0, 0)))
    res = _soft_embed(tok_t, wte_weight, softp)
    return jnp.transpose(res, (2, 4, 0, 1, 3)).reshape(B, S, D)
